# baseline (speedup 1.0000x reference)
import jax
import jax.numpy as jnp
from jax import lax
from jax.experimental import pallas as pl
from jax.experimental.pallas import tpu as pltpu
from jax.experimental.pallas import tpu_sc as plsc

N_FIELDS = 26
VOCAB = 100000
BATCH = 16384
EPS = 1e-5

NC = 2
LANES = 16

IDX_CHUNK = 4096
N_CHUNK = BATCH // IDX_CHUNK
UNROLL = 4
N_ACC = 2
HALF_V = VOCAB // 2


def _rsqrt16(x16):
    i = plsc.bitcast(x16, jnp.int32)
    i = jnp.int32(0x5F3759DF) - lax.shift_right_logical(i, 1)
    y = plsc.bitcast(i, jnp.float32)
    half = x16 * 0.5
    for _ in range(3):
        y = y * (1.5 - half * y * y)
    return y


def _sc_body(tables_hbm, idx_hbm, gamma_hbm, beta_hbm, out1_hbm, out2_hbm,
             table_v, idx_v0, idx_v1, feat_v, gb_v,
             sem_t, sem_i0, sem_i1, sem_o):
    wid = lax.axis_index("s") * NC + lax.axis_index("c")

    @pl.when(wid < N_FIELDS)
    def _():
        idx_bufs = (idx_v0, idx_v1)
        idx_sems = (sem_i0, sem_i1)

        cp_t = pltpu.async_copy(tables_hbm.at[wid], table_v, sem_t)
        cps = [
            pltpu.async_copy(
                idx_hbm.at[wid, pl.ds(c * IDX_CHUNK, IDX_CHUNK)],
                idx_bufs[c % 2], idx_sems[c % 2])
            for c in range(2)
        ]
        pltpu.sync_copy(gamma_hbm, gb_v.at[0])
        pltpu.sync_copy(beta_hbm, gb_v.at[1])
        cp_t.wait()

        zeros = jnp.zeros((LANES,), jnp.float32)
        zero_i16 = jnp.zeros((LANES,), jnp.int32)
        half_v16 = jnp.full((LANES,), HALF_V, jnp.int32)
        sixteen16 = jnp.full((LANES,), 16, jnp.int32)
        accs = (zeros,) * N_ACC
        accs2 = (zeros,) * N_ACC

        for c in range(N_CHUNK):
            cps[c].wait()
            buf = idx_bufs[c % 2]

            def body(i, carry, c=c, buf=buf):
                acc, acc2 = carry
                acc, acc2 = list(acc), list(acc2)
                base = pl.multiple_of(i * (LANES * UNROLL), LANES * UNROLL)
                for u in range(UNROLL):
                    off = base + u * LANES
                    idxs = buf[pl.ds(off, LANES)]
                    cond = idxs >= HALF_V
                    widx = idxs - lax.select(cond, half_v16, zero_i16)
                    sh = lax.select(cond, sixteen16, zero_i16)
                    word = plsc.load_gather(table_v, [widx])
                    bits = lax.shift_left(lax.shift_right_logical(word, sh), 16)
                    vals = plsc.bitcast(bits, jnp.float32)
                    feat_v[pl.ds(c * IDX_CHUNK + off, LANES)] = vals
                    acc[u % N_ACC] = acc[u % N_ACC] + vals
                    acc2[u % N_ACC] = acc2[u % N_ACC] + vals * vals
                return (tuple(acc), tuple(acc2))

            accs, accs2 = lax.fori_loop(
                0, IDX_CHUNK // (LANES * UNROLL), body, (accs, accs2))
            if c + 2 < N_CHUNK:
                cps.append(pltpu.async_copy(
                    idx_hbm.at[wid, pl.ds((c + 2) * IDX_CHUNK, IDX_CHUNK)],
                    idx_bufs[c % 2], idx_sems[c % 2]))

        acc = accs[0] + accs[1]
        acc2 = accs2[0] + accs2[1]

        inv_n = jnp.float32(1.0 / BATCH)
        mean = jnp.sum(acc) * inv_n
        var = jnp.sum(acc2) * inv_n - mean * mean
        mean16 = lax.broadcast_in_dim(mean, (LANES,), ())
        var16 = lax.broadcast_in_dim(var, (LANES,), ())
        inv16 = _rsqrt16(var16 + EPS)

        widv = lax.broadcast_in_dim(wid, (LANES,), ())
        g16 = plsc.load_gather(gb_v.at[0], [widv])
        b16 = plsc.load_gather(gb_v.at[1], [widv])
        scale16 = g16 * inv16
        shift16 = b16 - mean16 * scale16

        HALF = BATCH // 2

        def norm_half(h):
            def norm_body(i, carry):
                base = pl.multiple_of(
                    h * HALF + i * (LANES * UNROLL), LANES * UNROLL)
                for u in range(UNROLL):
                    off = base + u * LANES
                    feat_v[pl.ds(off, LANES)] = (
                        feat_v[pl.ds(off, LANES)] * scale16 + shift16)
                return carry
            lax.fori_loop(0, HALF // (LANES * UNROLL), norm_body, 0)

        norm_half(0)
        o1a = pltpu.async_copy(feat_v.at[pl.ds(0, HALF)],
                               out1_hbm.at[wid, pl.ds(0, HALF)], sem_o)
        o2a = pltpu.async_copy(feat_v.at[pl.ds(0, HALF)],
                               out2_hbm.at[wid, pl.ds(0, HALF)], sem_o)
        norm_half(1)
        o1b = pltpu.async_copy(feat_v.at[pl.ds(HALF, HALF)],
                               out1_hbm.at[wid, pl.ds(HALF, HALF)], sem_o)
        o2b = pltpu.async_copy(feat_v.at[pl.ds(HALF, HALF)],
                               out2_hbm.at[wid, pl.ds(HALF, HALF)], sem_o)
        o1a.wait()
        o2a.wait()
        o1b.wait()
        o2b.wait()


_sc_fused = pl.kernel(
    _sc_body,
    out_type=(jax.ShapeDtypeStruct((N_FIELDS, BATCH), jnp.float32),
              jax.ShapeDtypeStruct((N_FIELDS, BATCH), jnp.float32)),
    mesh=plsc.VectorSubcoreMesh(core_axis_name="c", subcore_axis_name="s"),
    compiler_params=pltpu.CompilerParams(needs_layout_passes=False),
    scratch_types=[
        pltpu.VMEM((VOCAB // 2,), jnp.int32),
        pltpu.VMEM((IDX_CHUNK,), jnp.int32),
        pltpu.VMEM((IDX_CHUNK,), jnp.int32),
        pltpu.VMEM((BATCH,), jnp.float32),
        pltpu.VMEM((2, N_FIELDS), jnp.float32),
        pltpu.SemaphoreType.DMA,
        pltpu.SemaphoreType.DMA,
        pltpu.SemaphoreType.DMA,
        pltpu.SemaphoreType.DMA,
    ],
)


def kernel(input_data, first_tables, gamma, beta):
    t2 = first_tables.reshape(N_FIELDS, VOCAB)
    lo = lax.bitcast_convert_type(
        t2[:, :HALF_V].astype(jnp.bfloat16), jnp.uint16).astype(jnp.uint32)
    hi = lax.bitcast_convert_type(
        t2[:, HALF_V:].astype(jnp.bfloat16), jnp.uint16).astype(jnp.uint32)
    tables = lax.bitcast_convert_type(lo | (hi << 16), jnp.int32)
    y1, y2 = _sc_fused(tables, input_data, gamma, beta)
    return (y1.T, y2.T)


# speedup vs baseline: 1.1677x; 1.1677x over previous
import jax
import jax.numpy as jnp
from jax import lax
from jax.experimental import pallas as pl
from jax.experimental.pallas import tpu as pltpu
from jax.experimental.pallas import tpu_sc as plsc

N_FIELDS = 26
VOCAB = 100000
BATCH = 16384
EPS = 1e-5

NC = 2
LANES = 16

IDX_CHUNK = 8192
N_CHUNK = BATCH // IDX_CHUNK
UNROLL = 8
N_ACC = 4
HALF_V = VOCAB // 2


def _rsqrt16(x16):
    i = plsc.bitcast(x16, jnp.int32)
    i = jnp.int32(0x5F3759DF) - lax.shift_right_logical(i, 1)
    y = plsc.bitcast(i, jnp.float32)
    half = x16 * 0.5
    for _ in range(3):
        y = y * (1.5 - half * y * y)
    return y


def _sc_body(tables_hbm, idx_hbm, gamma_hbm, beta_hbm, out1_hbm, out2_hbm,
             table_v, idx_v0, idx_v1, feat_v, gb_v,
             sem_t, sem_i0, sem_i1, sem_o):
    wid = lax.axis_index("s") * NC + lax.axis_index("c")

    @pl.when(wid < N_FIELDS)
    def _():
        idx_bufs = (idx_v0, idx_v1)
        idx_sems = (sem_i0, sem_i1)

        cp_t = pltpu.async_copy(tables_hbm.at[wid], table_v, sem_t)
        cps = [
            pltpu.async_copy(
                idx_hbm.at[wid, pl.ds(c * IDX_CHUNK, IDX_CHUNK)],
                idx_bufs[c % 2], idx_sems[c % 2])
            for c in range(2)
        ]
        pltpu.sync_copy(gamma_hbm, gb_v.at[0])
        pltpu.sync_copy(beta_hbm, gb_v.at[1])
        cp_t.wait()

        zeros = jnp.zeros((LANES,), jnp.float32)
        zero_i16 = jnp.zeros((LANES,), jnp.int32)
        half_v16 = jnp.full((LANES,), HALF_V, jnp.int32)
        sixteen16 = jnp.full((LANES,), 16, jnp.int32)
        accs = (zeros,) * N_ACC
        accs2 = (zeros,) * N_ACC

        for c in range(N_CHUNK):
            cps[c].wait()
            buf = idx_bufs[c % 2]

            def body(i, carry, c=c, buf=buf):
                acc, acc2 = carry
                acc, acc2 = list(acc), list(acc2)
                base = pl.multiple_of(i * (LANES * UNROLL), LANES * UNROLL)
                for u in range(UNROLL):
                    off = base + u * LANES
                    idxs = buf[pl.ds(off, LANES)]
                    cond = idxs >= HALF_V
                    widx = idxs - lax.select(cond, half_v16, zero_i16)
                    sh = lax.select(cond, sixteen16, zero_i16)
                    word = plsc.load_gather(table_v, [widx])
                    bits = lax.shift_left(lax.shift_right_logical(word, sh), 16)
                    vals = plsc.bitcast(bits, jnp.float32)
                    feat_v[pl.ds(c * IDX_CHUNK + off, LANES)] = vals
                    acc[u % N_ACC] = acc[u % N_ACC] + vals
                    acc2[u % N_ACC] = acc2[u % N_ACC] + vals * vals
                return (tuple(acc), tuple(acc2))

            accs, accs2 = lax.fori_loop(
                0, IDX_CHUNK // (LANES * UNROLL), body, (accs, accs2))
            if c + 2 < N_CHUNK:
                cps.append(pltpu.async_copy(
                    idx_hbm.at[wid, pl.ds((c + 2) * IDX_CHUNK, IDX_CHUNK)],
                    idx_bufs[c % 2], idx_sems[c % 2]))

        acc = accs[0] + accs[1] + (accs[2] + accs[3])
        acc2 = accs2[0] + accs2[1] + (accs2[2] + accs2[3])

        inv_n = jnp.float32(1.0 / BATCH)
        mean = jnp.sum(acc) * inv_n
        var = jnp.sum(acc2) * inv_n - mean * mean
        mean16 = lax.broadcast_in_dim(mean, (LANES,), ())
        var16 = lax.broadcast_in_dim(var, (LANES,), ())
        inv16 = _rsqrt16(var16 + EPS)

        widv = lax.broadcast_in_dim(wid, (LANES,), ())
        g16 = plsc.load_gather(gb_v.at[0], [widv])
        b16 = plsc.load_gather(gb_v.at[1], [widv])
        scale16 = g16 * inv16
        shift16 = b16 - mean16 * scale16

        QUART = BATCH // 4

        def norm_quarter(q):
            def norm_body(i, carry):
                base = pl.multiple_of(
                    q * QUART + i * (LANES * UNROLL), LANES * UNROLL)
                for u in range(UNROLL):
                    off = base + u * LANES
                    feat_v[pl.ds(off, LANES)] = (
                        feat_v[pl.ds(off, LANES)] * scale16 + shift16)
                return carry
            lax.fori_loop(0, QUART // (LANES * UNROLL), norm_body, 0)

        outs = []
        for q in range(4):
            norm_quarter(q)
            outs.append(pltpu.async_copy(
                feat_v.at[pl.ds(q * QUART, QUART)],
                out1_hbm.at[wid, pl.ds(q * QUART, QUART)], sem_o))
            outs.append(pltpu.async_copy(
                feat_v.at[pl.ds(q * QUART, QUART)],
                out2_hbm.at[wid, pl.ds(q * QUART, QUART)], sem_o))
        for o in outs:
            o.wait()


_sc_fused = pl.kernel(
    _sc_body,
    out_type=(jax.ShapeDtypeStruct((N_FIELDS, BATCH), jnp.float32),
              jax.ShapeDtypeStruct((N_FIELDS, BATCH), jnp.float32)),
    mesh=plsc.VectorSubcoreMesh(core_axis_name="c", subcore_axis_name="s"),
    compiler_params=pltpu.CompilerParams(needs_layout_passes=False),
    scratch_types=[
        pltpu.VMEM((VOCAB // 2,), jnp.int32),
        pltpu.VMEM((IDX_CHUNK,), jnp.int32),
        pltpu.VMEM((IDX_CHUNK,), jnp.int32),
        pltpu.VMEM((BATCH,), jnp.float32),
        pltpu.VMEM((2, N_FIELDS), jnp.float32),
        pltpu.SemaphoreType.DMA,
        pltpu.SemaphoreType.DMA,
        pltpu.SemaphoreType.DMA,
        pltpu.SemaphoreType.DMA,
    ],
)


def kernel(input_data, first_tables, gamma, beta):
    t2 = first_tables.reshape(N_FIELDS, VOCAB)
    lo = lax.bitcast_convert_type(
        t2[:, :HALF_V].astype(jnp.bfloat16), jnp.uint16).astype(jnp.uint32)
    hi = lax.bitcast_convert_type(
        t2[:, HALF_V:].astype(jnp.bfloat16), jnp.uint16).astype(jnp.uint32)
    tables = lax.bitcast_convert_type(lo | (hi << 16), jnp.int32)
    y1, y2 = _sc_fused(tables, input_data, gamma, beta)
    return (y1.T, y2.T)


# speedup vs baseline: 1.2743x; 1.0913x over previous
import jax
import jax.numpy as jnp
from jax import lax
from jax.experimental import pallas as pl
from jax.experimental.pallas import tpu as pltpu
from jax.experimental.pallas import tpu_sc as plsc

N_FIELDS = 26
VOCAB = 100000
BATCH = 16384
EPS = 1e-5

NC = 2
LANES = 16

IDX_CHUNK = 8192
N_CHUNK = BATCH // IDX_CHUNK
UNROLL = 8
N_ACC = 4
HALF_V = VOCAB // 2


def _rsqrt16(x16):
    i = plsc.bitcast(x16, jnp.int32)
    i = jnp.int32(0x5F3759DF) - lax.shift_right_logical(i, 1)
    y = plsc.bitcast(i, jnp.float32)
    half = x16 * 0.5
    for _ in range(3):
        y = y * (1.5 - half * y * y)
    return y


def _sc_body(tables_hbm, idx_hbm, gamma_hbm, beta_hbm, out1_hbm, out2_hbm,
             table_v, idx_v0, idx_v1, feat_v, gb_v,
             sem_t, sem_i0, sem_i1, sem_o):
    wid = lax.axis_index("s") * NC + lax.axis_index("c")

    @pl.when(wid < N_FIELDS)
    def _():
        idx_bufs = (idx_v0, idx_v1)
        idx_sems = (sem_i0, sem_i1)

        cp_t = pltpu.async_copy(tables_hbm.at[wid], table_v, sem_t)
        cps = [
            pltpu.async_copy(
                idx_hbm.at[wid, pl.ds(c * IDX_CHUNK, IDX_CHUNK)],
                idx_bufs[c % 2], idx_sems[c % 2])
            for c in range(2)
        ]
        pltpu.sync_copy(gamma_hbm, gb_v.at[0])
        pltpu.sync_copy(beta_hbm, gb_v.at[1])
        cp_t.wait()

        zeros = jnp.zeros((LANES,), jnp.float32)
        zero_i16 = jnp.zeros((LANES,), jnp.int32)
        half_v16 = jnp.full((LANES,), HALF_V, jnp.int32)
        sixteen16 = jnp.full((LANES,), 16, jnp.int32)
        accs = (zeros,) * N_ACC
        accs2 = (zeros,) * N_ACC

        for c in range(N_CHUNK):
            cps[c].wait()
            buf = idx_bufs[c % 2]

            @plsc.parallel_loop(0, IDX_CHUNK // (LANES * UNROLL),
                                carry=(accs, accs2))
            def _gather(i, carry, c=c, buf=buf):
                acc, acc2 = carry
                acc, acc2 = list(acc), list(acc2)
                base = pl.multiple_of(i * (LANES * UNROLL), LANES * UNROLL)
                for u in range(UNROLL):
                    off = base + u * LANES
                    idxs = buf[pl.ds(off, LANES)]
                    cond = idxs >= HALF_V
                    widx = idxs - lax.select(cond, half_v16, zero_i16)
                    sh = lax.select(cond, sixteen16, zero_i16)
                    word = plsc.load_gather(table_v, [widx])
                    bits = lax.shift_left(lax.shift_right_logical(word, sh), 16)
                    vals = plsc.bitcast(bits, jnp.float32)
                    feat_v[pl.ds(c * IDX_CHUNK + off, LANES)] = vals
                    acc[u % N_ACC] = acc[u % N_ACC] + vals
                    acc2[u % N_ACC] = acc2[u % N_ACC] + vals * vals
                return (tuple(acc), tuple(acc2))

            accs, accs2 = _gather
            if c + 2 < N_CHUNK:
                cps.append(pltpu.async_copy(
                    idx_hbm.at[wid, pl.ds((c + 2) * IDX_CHUNK, IDX_CHUNK)],
                    idx_bufs[c % 2], idx_sems[c % 2]))

        acc = accs[0] + accs[1] + (accs[2] + accs[3])
        acc2 = accs2[0] + accs2[1] + (accs2[2] + accs2[3])

        inv_n = jnp.float32(1.0 / BATCH)
        mean = jnp.sum(acc) * inv_n
        var = jnp.sum(acc2) * inv_n - mean * mean
        mean16 = lax.broadcast_in_dim(mean, (LANES,), ())
        var16 = lax.broadcast_in_dim(var, (LANES,), ())
        inv16 = _rsqrt16(var16 + EPS)

        widv = lax.broadcast_in_dim(wid, (LANES,), ())
        g16 = plsc.load_gather(gb_v.at[0], [widv])
        b16 = plsc.load_gather(gb_v.at[1], [widv])
        scale16 = g16 * inv16
        shift16 = b16 - mean16 * scale16

        QUART = BATCH // 4

        def norm_quarter(q):
            @plsc.parallel_loop(0, QUART // LANES, unroll=UNROLL)
            def _norm(i):
                off = q * QUART + i * LANES
                feat_v[pl.ds(off, LANES)] = (
                    feat_v[pl.ds(off, LANES)] * scale16 + shift16)

        outs = []
        for q in range(4):
            norm_quarter(q)
            outs.append(pltpu.async_copy(
                feat_v.at[pl.ds(q * QUART, QUART)],
                out1_hbm.at[wid, pl.ds(q * QUART, QUART)], sem_o))
            outs.append(pltpu.async_copy(
                feat_v.at[pl.ds(q * QUART, QUART)],
                out2_hbm.at[wid, pl.ds(q * QUART, QUART)], sem_o))
        for o in outs:
            o.wait()


_sc_fused = pl.kernel(
    _sc_body,
    out_type=(jax.ShapeDtypeStruct((N_FIELDS, BATCH), jnp.float32),
              jax.ShapeDtypeStruct((N_FIELDS, BATCH), jnp.float32)),
    mesh=plsc.VectorSubcoreMesh(core_axis_name="c", subcore_axis_name="s"),
    compiler_params=pltpu.CompilerParams(needs_layout_passes=False),
    scratch_types=[
        pltpu.VMEM((VOCAB // 2,), jnp.int32),
        pltpu.VMEM((IDX_CHUNK,), jnp.int32),
        pltpu.VMEM((IDX_CHUNK,), jnp.int32),
        pltpu.VMEM((BATCH,), jnp.float32),
        pltpu.VMEM((2, N_FIELDS), jnp.float32),
        pltpu.SemaphoreType.DMA,
        pltpu.SemaphoreType.DMA,
        pltpu.SemaphoreType.DMA,
        pltpu.SemaphoreType.DMA,
    ],
)


def kernel(input_data, first_tables, gamma, beta):
    t2 = first_tables.reshape(N_FIELDS, VOCAB)
    lo = lax.bitcast_convert_type(
        t2[:, :HALF_V].astype(jnp.bfloat16), jnp.uint16).astype(jnp.uint32)
    hi = lax.bitcast_convert_type(
        t2[:, HALF_V:].astype(jnp.bfloat16), jnp.uint16).astype(jnp.uint32)
    tables = lax.bitcast_convert_type(lo | (hi << 16), jnp.int32)
    y1, y2 = _sc_fused(tables, input_data, gamma, beta)
    return (y1.T, y2.T)
